# Initial kernel scaffold; baseline (speedup 1.0000x reference)
import jax
import jax.numpy as jnp
from jax.experimental import pallas as pl


def kernel(x, edge_index, W_l, b_l, W_r):
    raise NotImplementedError("write your pallas kernel here")


# speedup vs baseline: 3.4336x; 3.4336x over previous
import jax
import jax.numpy as jnp
from jax import lax
from jax.experimental import pallas as pl
from jax.experimental.pallas import tpu as pltpu
from jax.experimental.pallas import tpu_sc as plsc

N_NODES = 10000
D = 256
HALF = 128
E = 160000

NUM_SUBCORES = 16
CHUNK = 128
OUTER = 10
EDGES_PER_TILE = CHUNK * 8 * OUTER
E_PAD = EDGES_PER_TILE * NUM_SUBCORES
N_ACC = 10240
ROWS_PER_TILE = N_ACC // NUM_SUBCORES
COUTER = 5


def _sc_agg_kernel(x0_hbm, x1_hbm, src_hbm, dst_hbm,
                   agg0_hbm, agg1_hbm,
                   src_idx, dst_idx, rows, acc, sem):
    c = lax.axis_index("c")
    s = lax.axis_index("s")

    zeros16 = jnp.zeros((16,), jnp.float32)

    def _fill_rows(i, _):
        for k in range(HALF // 16):
            rows[i, pl.ds(k * 16, 16)] = zeros16
        return 0
    lax.fori_loop(0, CHUNK, _fill_rows, 0)

    rbase = s * ROWS_PER_TILE
    for z in range(ROWS_PER_TILE // CHUNK):
        pltpu.sync_copy(rows, acc.at[pl.ds(rbase + z * CHUNK, CHUNK)])

    plsc.subcore_barrier()

    def _outer(j, _):
        pltpu.sync_copy(src_hbm.at[s, j], src_idx)
        pltpu.sync_copy(dst_hbm.at[s, j], dst_idx)

        def _inner(k, _):
            @pl.when(c == 0)
            def _():
                pltpu.async_copy(x0_hbm.at[src_idx.at[k]], rows, sem).wait()

            @pl.when(c == 1)
            def _():
                pltpu.async_copy(x1_hbm.at[src_idx.at[k]], rows, sem).wait()

            pltpu.sync_copy(rows, acc.at[dst_idx.at[k]], add=True)
            return 0

        lax.fori_loop(0, 8, _inner, 0)
        return 0

    lax.fori_loop(0, OUTER, _outer, 0)

    plsc.subcore_barrier()

    for z in range(ROWS_PER_TILE // CHUNK):
        pltpu.sync_copy(acc.at[pl.ds(rbase + z * CHUNK, CHUNK)], rows)

        @pl.when(c == 0)
        def _():
            pltpu.sync_copy(rows, agg0_hbm.at[s, pl.ds(z * CHUNK, CHUNK)])

        @pl.when(c == 1)
        def _():
            pltpu.sync_copy(rows, agg1_hbm.at[s, pl.ds(z * CHUNK, CHUNK)])


def _sc_count_kernel(dst_hbm, cnt0_hbm, cnt1_hbm,
                     dst_idx, rows, cacc, sem):
    c = lax.axis_index("c")
    s = lax.axis_index("s")

    zeros16 = jnp.zeros((16,), jnp.float32)
    ones16 = jnp.ones((16,), jnp.float32)

    def _fill_zeros(i, _):
        for k in range(HALF // 16):
            rows[i, pl.ds(k * 16, 16)] = zeros16
        return 0
    lax.fori_loop(0, CHUNK, _fill_zeros, 0)

    rbase = s * ROWS_PER_TILE
    for z in range(ROWS_PER_TILE // CHUNK):
        pltpu.sync_copy(rows, cacc.at[pl.ds(rbase + z * CHUNK, CHUNK)])

    def _fill_ones(i, _):
        for k in range(HALF // 16):
            rows[i, pl.ds(k * 16, 16)] = ones16
        return 0
    lax.fori_loop(0, CHUNK, _fill_ones, 0)

    plsc.subcore_barrier()

    def _outer(j, _):
        pltpu.sync_copy(dst_hbm.at[c, s, j], dst_idx)

        def _inner(k, _):
            pltpu.sync_copy(rows, cacc.at[dst_idx.at[k]], add=True)
            return 0

        lax.fori_loop(0, 8, _inner, 0)
        return 0

    lax.fori_loop(0, COUTER, _outer, 0)

    plsc.subcore_barrier()

    for z in range(ROWS_PER_TILE // CHUNK):
        pltpu.sync_copy(cacc.at[pl.ds(rbase + z * CHUNK, CHUNK)], rows)

        @pl.when(c == 0)
        def _():
            pltpu.sync_copy(rows, cnt0_hbm.at[s, pl.ds(z * CHUNK, CHUNK)])

        @pl.when(c == 1)
        def _():
            pltpu.sync_copy(rows, cnt1_hbm.at[s, pl.ds(z * CHUNK, CHUNK)])


@jax.jit
def _sc_aggregate(x0, x1, src4d, dst4d, dst5d):
    mesh = plsc.VectorSubcoreMesh(core_axis_name="c", subcore_axis_name="s")
    agg = pl.kernel(
        _sc_agg_kernel,
        mesh=mesh,
        out_type=[
            jax.ShapeDtypeStruct((NUM_SUBCORES, ROWS_PER_TILE, HALF), jnp.float32),
            jax.ShapeDtypeStruct((NUM_SUBCORES, ROWS_PER_TILE, HALF), jnp.float32),
        ],
        scratch_types=[
            pltpu.VMEM((8, CHUNK), jnp.int32),
            pltpu.VMEM((8, CHUNK), jnp.int32),
            pltpu.VMEM((CHUNK, HALF), jnp.float32),
            pltpu.VMEM_SHARED((N_ACC, HALF), jnp.float32),
            pltpu.SemaphoreType.DMA,
        ],
    )
    cnt = pl.kernel(
        _sc_count_kernel,
        mesh=mesh,
        out_type=[
            jax.ShapeDtypeStruct((NUM_SUBCORES, ROWS_PER_TILE, HALF), jnp.float32),
            jax.ShapeDtypeStruct((NUM_SUBCORES, ROWS_PER_TILE, HALF), jnp.float32),
        ],
        scratch_types=[
            pltpu.VMEM((8, CHUNK), jnp.int32),
            pltpu.VMEM((CHUNK, HALF), jnp.float32),
            pltpu.VMEM_SHARED((N_ACC, HALF), jnp.float32),
            pltpu.SemaphoreType.DMA,
        ],
    )
    agg0, agg1 = agg(x0, x1, src4d, dst4d)
    cnt0, cnt1 = cnt(dst5d)
    return agg0, agg1, cnt0, cnt1


def _tc_dense_kernel(x_ref, a0_ref, a1_ref, c0_ref, c1_ref,
                     wl_ref, bl_ref, wr_ref, out_ref):
    cnt = c0_ref[:, 0:1] + c1_ref[:, 0:1]
    denom = jnp.maximum(cnt, 1.0)
    agg = jnp.concatenate([a0_ref[...], a1_ref[...]], axis=1) / denom
    dn = (((1,), (1,)), ((), ()))
    out = lax.dot_general(agg, wl_ref[...], dn,
                          preferred_element_type=jnp.float32)
    out += lax.dot_general(x_ref[...], wr_ref[...], dn,
                           preferred_element_type=jnp.float32)
    out += bl_ref[...]
    out_ref[...] = jnp.maximum(out, 0.0)


@jax.jit
def _tc_dense(x, agg0, agg1, cnt0, cnt1, W_l, b_l2d, W_r):
    grid = 10
    bn = N_NODES // grid
    return pl.pallas_call(
        _tc_dense_kernel,
        grid=(grid,),
        in_specs=[
            pl.BlockSpec((bn, D), lambda i: (i, 0)),
            pl.BlockSpec((bn, HALF), lambda i: (i, 0)),
            pl.BlockSpec((bn, HALF), lambda i: (i, 0)),
            pl.BlockSpec((bn, HALF), lambda i: (i, 0)),
            pl.BlockSpec((bn, HALF), lambda i: (i, 0)),
            pl.BlockSpec((D, D), lambda i: (0, 0)),
            pl.BlockSpec((1, D), lambda i: (0, 0)),
            pl.BlockSpec((D, D), lambda i: (0, 0)),
        ],
        out_specs=pl.BlockSpec((bn, D), lambda i: (i, 0)),
        out_shape=jax.ShapeDtypeStruct((N_NODES, D), jnp.float32),
    )(x, agg0, agg1, cnt0, cnt1, W_l, b_l2d, W_r)


def kernel(x, edge_index, W_l, b_l, W_r):
    ei = edge_index.astype(jnp.int32)
    npad = E_PAD - E
    src = jnp.concatenate([ei[0], jnp.zeros((npad,), jnp.int32)])
    dst = jnp.concatenate([ei[1], jnp.full((npad,), N_NODES, jnp.int32)])
    src4d = src.reshape(NUM_SUBCORES, OUTER, 8, CHUNK)
    dst4d = dst.reshape(NUM_SUBCORES, OUTER, 8, CHUNK)
    dst5d = dst.reshape(2, NUM_SUBCORES, COUTER, 8, CHUNK)
    x0 = x[:, :HALF]
    x1 = x[:, HALF:]
    agg0, agg1, cnt0, cnt1 = _sc_aggregate(x0, x1, src4d, dst4d, dst5d)
    agg0 = agg0.reshape(N_ACC, HALF)[:N_NODES]
    agg1 = agg1.reshape(N_ACC, HALF)[:N_NODES]
    cnt0 = cnt0.reshape(N_ACC, HALF)[:N_NODES]
    cnt1 = cnt1.reshape(N_ACC, HALF)[:N_NODES]
    return _tc_dense(x, agg0, agg1, cnt0, cnt1, W_l, b_l.reshape(1, D), W_r)


# speedup vs baseline: 3.5060x; 1.0211x over previous
import jax
import jax.numpy as jnp
from jax import lax
from jax.experimental import pallas as pl
from jax.experimental.pallas import tpu as pltpu
from jax.experimental.pallas import tpu_sc as plsc

N_NODES = 10000
D = 256
HALF = 128
E = 160000

NUM_SUBCORES = 16
CHUNK = 128
OUTER = 10
EDGES_PER_TILE = CHUNK * 8 * OUTER
E_PAD = EDGES_PER_TILE * NUM_SUBCORES
N_ACC = 10240
ROWS_PER_TILE = N_ACC // NUM_SUBCORES
COUTER = 5


def _sc_agg_kernel(xh_hbm, src_hbm, dst_hbm,
                   agg0_hbm, agg1_hbm,
                   src_idx, dst_idx, rows_a, rows_b, acc,
                   gsem_a, gsem_b, ssem_a, ssem_b):
    c = lax.axis_index("c")
    s = lax.axis_index("s")

    zeros16 = jnp.zeros((16,), jnp.float32)

    def _fill_rows(i, _):
        for k in range(HALF // 16):
            rows_a[i, pl.ds(k * 16, 16)] = zeros16
        return 0
    lax.fori_loop(0, CHUNK, _fill_rows, 0)

    rbase = s * ROWS_PER_TILE
    for z in range(ROWS_PER_TILE // CHUNK):
        pltpu.sync_copy(rows_a, acc.at[pl.ds(rbase + z * CHUNK, CHUNK)])

    plsc.subcore_barrier()

    xc = xh_hbm.at[c]
    bufs = (rows_a, rows_b)
    gsems = (gsem_a, gsem_b)
    ssems = (ssem_a, ssem_b)

    def _outer(j, _):
        pltpu.sync_copy(src_hbm.at[s, j], src_idx)
        pltpu.sync_copy(dst_hbm.at[s, j], dst_idx)

        gd = {}
        sd = {}
        gd[0] = pltpu.async_copy(xc.at[src_idx.at[0]], rows_a, gsem_a)
        for k in range(8):
            cur = bufs[k % 2]
            gd[k].wait()
            if k + 1 < 8:
                if k - 1 >= 0:
                    sd[k - 1].wait()
                gd[k + 1] = pltpu.async_copy(
                    xc.at[src_idx.at[k + 1]], bufs[(k + 1) % 2],
                    gsems[(k + 1) % 2])
            sd[k] = pltpu.async_copy(cur, acc.at[dst_idx.at[k]],
                                     ssems[k % 2], add=True)
        sd[6].wait()
        sd[7].wait()
        return 0

    lax.fori_loop(0, OUTER, _outer, 0)

    plsc.subcore_barrier()

    for z in range(ROWS_PER_TILE // CHUNK):
        pltpu.sync_copy(acc.at[pl.ds(rbase + z * CHUNK, CHUNK)], rows_a)

        @pl.when(c == 0)
        def _():
            pltpu.sync_copy(rows_a, agg0_hbm.at[s, pl.ds(z * CHUNK, CHUNK)])

        @pl.when(c == 1)
        def _():
            pltpu.sync_copy(rows_a, agg1_hbm.at[s, pl.ds(z * CHUNK, CHUNK)])


def _sc_count_kernel(dst_hbm, cnt0_hbm, cnt1_hbm,
                     dst_idx, rows, cacc, csem):
    c = lax.axis_index("c")
    s = lax.axis_index("s")

    zeros16 = jnp.zeros((16,), jnp.float32)
    ones16 = jnp.ones((16,), jnp.float32)

    def _fill_zeros(i, _):
        for k in range(HALF // 16):
            rows[i, pl.ds(k * 16, 16)] = zeros16
        return 0
    lax.fori_loop(0, CHUNK, _fill_zeros, 0)

    rbase = s * ROWS_PER_TILE
    for z in range(ROWS_PER_TILE // CHUNK):
        pltpu.sync_copy(rows, cacc.at[pl.ds(rbase + z * CHUNK, CHUNK)])

    def _fill_ones(i, _):
        for k in range(HALF // 16):
            rows[i, pl.ds(k * 16, 16)] = ones16
        return 0
    lax.fori_loop(0, CHUNK, _fill_ones, 0)

    plsc.subcore_barrier()

    def _outer(j, _):
        pltpu.sync_copy(dst_hbm.at[c, s, j], dst_idx)
        sds = [pltpu.async_copy(rows, cacc.at[dst_idx.at[k]], csem, add=True)
               for k in range(8)]
        for sd in sds:
            sd.wait()
        return 0

    lax.fori_loop(0, COUTER, _outer, 0)

    plsc.subcore_barrier()

    for z in range(ROWS_PER_TILE // CHUNK):
        pltpu.sync_copy(cacc.at[pl.ds(rbase + z * CHUNK, CHUNK)], rows)

        @pl.when(c == 0)
        def _():
            pltpu.sync_copy(rows, cnt0_hbm.at[s, pl.ds(z * CHUNK, CHUNK)])

        @pl.when(c == 1)
        def _():
            pltpu.sync_copy(rows, cnt1_hbm.at[s, pl.ds(z * CHUNK, CHUNK)])


@jax.jit
def _sc_aggregate(xh, src4d, dst4d, dst5d):
    mesh = plsc.VectorSubcoreMesh(core_axis_name="c", subcore_axis_name="s")
    agg = pl.kernel(
        _sc_agg_kernel,
        mesh=mesh,
        out_type=[
            jax.ShapeDtypeStruct((NUM_SUBCORES, ROWS_PER_TILE, HALF), jnp.float32),
            jax.ShapeDtypeStruct((NUM_SUBCORES, ROWS_PER_TILE, HALF), jnp.float32),
        ],
        scratch_types=[
            pltpu.VMEM((8, CHUNK), jnp.int32),
            pltpu.VMEM((8, CHUNK), jnp.int32),
            pltpu.VMEM((CHUNK, HALF), jnp.float32),
            pltpu.VMEM((CHUNK, HALF), jnp.float32),
            pltpu.VMEM_SHARED((N_ACC, HALF), jnp.float32),
            pltpu.SemaphoreType.DMA,
            pltpu.SemaphoreType.DMA,
            pltpu.SemaphoreType.DMA,
            pltpu.SemaphoreType.DMA,
        ],
    )
    cnt = pl.kernel(
        _sc_count_kernel,
        mesh=mesh,
        out_type=[
            jax.ShapeDtypeStruct((NUM_SUBCORES, ROWS_PER_TILE, HALF), jnp.float32),
            jax.ShapeDtypeStruct((NUM_SUBCORES, ROWS_PER_TILE, HALF), jnp.float32),
        ],
        scratch_types=[
            pltpu.VMEM((8, CHUNK), jnp.int32),
            pltpu.VMEM((CHUNK, HALF), jnp.float32),
            pltpu.VMEM_SHARED((N_ACC, HALF), jnp.float32),
            pltpu.SemaphoreType.DMA,
        ],
    )
    agg0, agg1 = agg(xh, src4d, dst4d)
    cnt0, cnt1 = cnt(dst5d)
    return agg0, agg1, cnt0, cnt1


def _tc_dense_kernel(x_ref, a0_ref, a1_ref, c0_ref, c1_ref,
                     wl_ref, bl_ref, wr_ref, out_ref):
    cnt = c0_ref[:, 0:1] + c1_ref[:, 0:1]
    denom = jnp.maximum(cnt, 1.0)
    agg = jnp.concatenate([a0_ref[...], a1_ref[...]], axis=1) / denom
    dn = (((1,), (1,)), ((), ()))
    out = lax.dot_general(agg, wl_ref[...], dn,
                          preferred_element_type=jnp.float32)
    out += lax.dot_general(x_ref[...], wr_ref[...], dn,
                           preferred_element_type=jnp.float32)
    out += bl_ref[...]
    out_ref[...] = jnp.maximum(out, 0.0)


@jax.jit
def _tc_dense(x, agg0, agg1, cnt0, cnt1, W_l, b_l2d, W_r):
    grid = 10
    bn = N_NODES // grid
    return pl.pallas_call(
        _tc_dense_kernel,
        grid=(grid,),
        in_specs=[
            pl.BlockSpec((bn, D), lambda i: (i, 0)),
            pl.BlockSpec((bn, HALF), lambda i: (i, 0)),
            pl.BlockSpec((bn, HALF), lambda i: (i, 0)),
            pl.BlockSpec((bn, HALF), lambda i: (i, 0)),
            pl.BlockSpec((bn, HALF), lambda i: (i, 0)),
            pl.BlockSpec((D, D), lambda i: (0, 0)),
            pl.BlockSpec((1, D), lambda i: (0, 0)),
            pl.BlockSpec((D, D), lambda i: (0, 0)),
        ],
        out_specs=pl.BlockSpec((bn, D), lambda i: (i, 0)),
        out_shape=jax.ShapeDtypeStruct((N_NODES, D), jnp.float32),
    )(x, agg0, agg1, cnt0, cnt1, W_l, b_l2d, W_r)


def kernel(x, edge_index, W_l, b_l, W_r):
    ei = edge_index.astype(jnp.int32)
    npad = E_PAD - E
    src = jnp.concatenate([ei[0], jnp.zeros((npad,), jnp.int32)])
    dst = jnp.concatenate([ei[1], jnp.full((npad,), N_NODES, jnp.int32)])
    src4d = src.reshape(NUM_SUBCORES, OUTER, 8, CHUNK)
    dst4d = dst.reshape(NUM_SUBCORES, OUTER, 8, CHUNK)
    dst5d = dst.reshape(2, NUM_SUBCORES, COUTER, 8, CHUNK)
    xh = jnp.stack([x[:, :HALF], x[:, HALF:]])
    agg0, agg1, cnt0, cnt1 = _sc_aggregate(xh, src4d, dst4d, dst5d)
    agg0 = agg0.reshape(N_ACC, HALF)[:N_NODES]
    agg1 = agg1.reshape(N_ACC, HALF)[:N_NODES]
    cnt0 = cnt0.reshape(N_ACC, HALF)[:N_NODES]
    cnt1 = cnt1.reshape(N_ACC, HALF)[:N_NODES]
    return _tc_dense(x, agg0, agg1, cnt0, cnt1, W_l, b_l.reshape(1, D), W_r)


# speedup vs baseline: 3.7978x; 1.0832x over previous
import jax
import jax.numpy as jnp
from jax import lax
from jax.experimental import pallas as pl
from jax.experimental.pallas import tpu as pltpu
from jax.experimental.pallas import tpu_sc as plsc

N_NODES = 10000
D = 256
HALF = 128
E = 160000

NUM_SUBCORES = 16
CHUNK = 128
OUTER = 10
EDGES_PER_TILE = CHUNK * 8 * OUTER
E_PAD = EDGES_PER_TILE * NUM_SUBCORES
N_ACC = 10240
ROWS_PER_TILE = N_ACC // NUM_SUBCORES
COUTER = 5


def _sc_agg_kernel(x_hbm, src_hbm, dst_hbm,
                   agg0_hbm, agg1_hbm,
                   src_idx, dst_idx, rows_a, rows_b, acc,
                   gsem_a, gsem_b, ssem_a, ssem_b, isem):
    c = lax.axis_index("c")
    s = lax.axis_index("s")

    zeros16 = jnp.zeros((16,), jnp.float32)

    def _fill_rows(i, _):
        for k in range(HALF // 16):
            rows_a[i, pl.ds(k * 16, 16)] = zeros16
        return 0
    lax.fori_loop(0, CHUNK, _fill_rows, 0)

    rbase = s * ROWS_PER_TILE
    for z in range(ROWS_PER_TILE // CHUNK):
        pltpu.sync_copy(rows_a, acc.at[pl.ds(rbase + z * CHUNK, CHUNK)])

    plsc.subcore_barrier()

    coff = pl.multiple_of(c * HALF, HALF)
    xc = x_hbm.at[:, pl.ds(coff, HALF)]
    bufs = (rows_a, rows_b)
    gsems = (gsem_a, gsem_b)
    ssems = (ssem_a, ssem_b)

    pltpu.sync_copy(src_hbm.at[s, 0], src_idx.at[0])
    pltpu.sync_copy(dst_hbm.at[s, 0], dst_idx.at[0])

    def _outer(j, _):
        jb = lax.rem(j, 2)
        sidx = src_idx.at[jb]
        didx = dst_idx.at[jb]

        gd = {}
        sd = {}
        gd[0] = pltpu.async_copy(xc.at[sidx.at[0]], rows_a, gsem_a)
        gd[1] = pltpu.async_copy(xc.at[sidx.at[1]], rows_b, gsem_b)

        nj = j + 1
        nid = None

        @pl.when(nj < OUTER)
        def _():
            jn = lax.rem(nj, 2)
            pltpu.async_copy(src_hbm.at[s, nj], src_idx.at[jn], isem).wait()
            pltpu.async_copy(dst_hbm.at[s, nj], dst_idx.at[jn], isem).wait()

        for k in range(8):
            cur = bufs[k % 2]
            gd[k].wait()
            sd[k] = pltpu.async_copy(cur, acc.at[didx.at[k]],
                                     ssems[k % 2], add=True)
            if k + 2 < 8:
                sd[k].wait()
                gd[k + 2] = pltpu.async_copy(xc.at[sidx.at[k + 2]], cur,
                                             gsems[k % 2])
        sd[6].wait()
        sd[7].wait()
        return 0

    lax.fori_loop(0, OUTER, _outer, 0)

    plsc.subcore_barrier()

    for z in range(ROWS_PER_TILE // CHUNK):
        pltpu.sync_copy(acc.at[pl.ds(rbase + z * CHUNK, CHUNK)], rows_a)

        @pl.when(c == 0)
        def _():
            pltpu.sync_copy(rows_a, agg0_hbm.at[s, pl.ds(z * CHUNK, CHUNK)])

        @pl.when(c == 1)
        def _():
            pltpu.sync_copy(rows_a, agg1_hbm.at[s, pl.ds(z * CHUNK, CHUNK)])


def _sc_count_kernel(dst_hbm, cnt0_hbm, cnt1_hbm,
                     dst_idx, rows, cacc, csem):
    c = lax.axis_index("c")
    s = lax.axis_index("s")

    zeros16 = jnp.zeros((16,), jnp.float32)
    ones16 = jnp.ones((16,), jnp.float32)

    def _fill_zeros(i, _):
        for k in range(HALF // 16):
            rows[i, pl.ds(k * 16, 16)] = zeros16
        return 0
    lax.fori_loop(0, CHUNK, _fill_zeros, 0)

    rbase = s * ROWS_PER_TILE
    for z in range(ROWS_PER_TILE // CHUNK):
        pltpu.sync_copy(rows, cacc.at[pl.ds(rbase + z * CHUNK, CHUNK)])

    def _fill_ones(i, _):
        for k in range(HALF // 16):
            rows[i, pl.ds(k * 16, 16)] = ones16
        return 0
    lax.fori_loop(0, CHUNK, _fill_ones, 0)

    plsc.subcore_barrier()

    def _outer(j, _):
        pltpu.sync_copy(dst_hbm.at[c, s, j], dst_idx)
        sds = [pltpu.async_copy(rows, cacc.at[dst_idx.at[k]], csem, add=True)
               for k in range(8)]
        for sd in sds:
            sd.wait()
        return 0

    lax.fori_loop(0, COUTER, _outer, 0)

    plsc.subcore_barrier()

    for z in range(ROWS_PER_TILE // CHUNK):
        pltpu.sync_copy(cacc.at[pl.ds(rbase + z * CHUNK, CHUNK)], rows)

        @pl.when(c == 0)
        def _():
            pltpu.sync_copy(rows, cnt0_hbm.at[s, pl.ds(z * CHUNK, CHUNK)])

        @pl.when(c == 1)
        def _():
            pltpu.sync_copy(rows, cnt1_hbm.at[s, pl.ds(z * CHUNK, CHUNK)])


@jax.jit
def _sc_aggregate(x, src4d, dst4d, dst5d):
    mesh = plsc.VectorSubcoreMesh(core_axis_name="c", subcore_axis_name="s")
    agg = pl.kernel(
        _sc_agg_kernel,
        mesh=mesh,
        out_type=[
            jax.ShapeDtypeStruct((NUM_SUBCORES, ROWS_PER_TILE, HALF), jnp.float32),
            jax.ShapeDtypeStruct((NUM_SUBCORES, ROWS_PER_TILE, HALF), jnp.float32),
        ],
        scratch_types=[
            pltpu.VMEM((2, 8, CHUNK), jnp.int32),
            pltpu.VMEM((2, 8, CHUNK), jnp.int32),
            pltpu.VMEM((CHUNK, HALF), jnp.float32),
            pltpu.VMEM((CHUNK, HALF), jnp.float32),
            pltpu.VMEM_SHARED((N_ACC, HALF), jnp.float32),
            pltpu.SemaphoreType.DMA,
            pltpu.SemaphoreType.DMA,
            pltpu.SemaphoreType.DMA,
            pltpu.SemaphoreType.DMA,
            pltpu.SemaphoreType.DMA,
        ],
    )
    cnt = pl.kernel(
        _sc_count_kernel,
        mesh=mesh,
        out_type=[
            jax.ShapeDtypeStruct((NUM_SUBCORES, ROWS_PER_TILE, HALF), jnp.float32),
            jax.ShapeDtypeStruct((NUM_SUBCORES, ROWS_PER_TILE, HALF), jnp.float32),
        ],
        scratch_types=[
            pltpu.VMEM((8, CHUNK), jnp.int32),
            pltpu.VMEM((CHUNK, HALF), jnp.float32),
            pltpu.VMEM_SHARED((N_ACC, HALF), jnp.float32),
            pltpu.SemaphoreType.DMA,
        ],
    )
    agg0, agg1 = agg(x, src4d, dst4d)
    cnt0, cnt1 = cnt(dst5d)
    return agg0, agg1, cnt0, cnt1


def _tc_dense_kernel(x_ref, a0_ref, a1_ref, c0_ref, c1_ref,
                     wl_ref, bl_ref, wr_ref, out_ref):
    cnt = c0_ref[:, 0:1] + c1_ref[:, 0:1]
    denom = jnp.maximum(cnt, 1.0)
    agg = jnp.concatenate([a0_ref[...], a1_ref[...]], axis=1) / denom
    dn = (((1,), (1,)), ((), ()))
    out = lax.dot_general(agg, wl_ref[...], dn,
                          preferred_element_type=jnp.float32)
    out += lax.dot_general(x_ref[...], wr_ref[...], dn,
                           preferred_element_type=jnp.float32)
    out += bl_ref[...]
    out_ref[...] = jnp.maximum(out, 0.0)


@jax.jit
def _tc_dense(x, agg0, agg1, cnt0, cnt1, W_l, b_l2d, W_r):
    grid = 10
    bn = N_NODES // grid
    return pl.pallas_call(
        _tc_dense_kernel,
        grid=(grid,),
        in_specs=[
            pl.BlockSpec((bn, D), lambda i: (i, 0)),
            pl.BlockSpec((bn, HALF), lambda i: (i, 0)),
            pl.BlockSpec((bn, HALF), lambda i: (i, 0)),
            pl.BlockSpec((bn, HALF), lambda i: (i, 0)),
            pl.BlockSpec((bn, HALF), lambda i: (i, 0)),
            pl.BlockSpec((D, D), lambda i: (0, 0)),
            pl.BlockSpec((1, D), lambda i: (0, 0)),
            pl.BlockSpec((D, D), lambda i: (0, 0)),
        ],
        out_specs=pl.BlockSpec((bn, D), lambda i: (i, 0)),
        out_shape=jax.ShapeDtypeStruct((N_NODES, D), jnp.float32),
    )(x, agg0, agg1, cnt0, cnt1, W_l, b_l2d, W_r)


def kernel(x, edge_index, W_l, b_l, W_r):
    ei = edge_index.astype(jnp.int32)
    npad = E_PAD - E
    src = jnp.concatenate([ei[0], jnp.zeros((npad,), jnp.int32)])
    dst = jnp.concatenate([ei[1], jnp.full((npad,), N_NODES, jnp.int32)])
    src4d = src.reshape(NUM_SUBCORES, OUTER, 8, CHUNK)
    dst4d = dst.reshape(NUM_SUBCORES, OUTER, 8, CHUNK)
    dst5d = dst.reshape(2, NUM_SUBCORES, COUTER, 8, CHUNK)
    agg0, agg1, cnt0, cnt1 = _sc_aggregate(x, src4d, dst4d, dst5d)
    agg0 = agg0.reshape(N_ACC, HALF)[:N_NODES]
    agg1 = agg1.reshape(N_ACC, HALF)[:N_NODES]
    cnt0 = cnt0.reshape(N_ACC, HALF)[:N_NODES]
    cnt1 = cnt1.reshape(N_ACC, HALF)[:N_NODES]
    return _tc_dense(x, agg0, agg1, cnt0, cnt1, W_l, b_l.reshape(1, D), W_r)


# speedup vs baseline: 3.8074x; 1.0025x over previous
import jax
import jax.numpy as jnp
from jax import lax
from jax.experimental import pallas as pl
from jax.experimental.pallas import tpu as pltpu
from jax.experimental.pallas import tpu_sc as plsc

N_NODES = 10000
D = 256
HALF = 128
E = 160000

NUM_SUBCORES = 16
CHUNK = 128
OUTER = 10
EDGES_PER_TILE = CHUNK * 8 * OUTER
E_PAD = EDGES_PER_TILE * NUM_SUBCORES
N_ACC = 10240
ROWS_PER_TILE = N_ACC // NUM_SUBCORES
COUTER = 5


def _sc_agg_kernel(x_hbm, src_hbm, dst_hbm,
                   agg0_hbm, agg1_hbm,
                   src_idx, dst_idx, rows_a, rows_b, acc,
                   gsem_a, gsem_b, gsem_c, gsem_d, ssem_a, ssem_b, isem):
    c = lax.axis_index("c")
    s = lax.axis_index("s")

    zeros16 = jnp.zeros((16,), jnp.float32)

    def _fill_rows(i, _):
        for k in range(HALF // 16):
            rows_a[i, pl.ds(k * 16, 16)] = zeros16
        return 0
    lax.fori_loop(0, CHUNK, _fill_rows, 0)

    rbase = s * ROWS_PER_TILE
    for z in range(ROWS_PER_TILE // CHUNK):
        pltpu.sync_copy(rows_a, acc.at[pl.ds(rbase + z * CHUNK, CHUNK)])

    plsc.subcore_barrier()

    coff = pl.multiple_of(c * HALF, HALF)
    xc = x_hbm.at[:, pl.ds(coff, HALF)]
    bufs = (rows_a, rows_b)
    gsems = (gsem_a, gsem_b)
    ssems = (ssem_a, ssem_b)

    pltpu.sync_copy(src_hbm.at[s, 0], src_idx.at[0])
    pltpu.sync_copy(dst_hbm.at[s, 0], dst_idx.at[0])

    halves = (rows_a.at[pl.ds(0, 64)], rows_a.at[pl.ds(64, 64)],
              rows_b.at[pl.ds(0, 64)], rows_b.at[pl.ds(64, 64)])
    gsems4 = (gsem_a, gsem_b, gsem_c, gsem_d)

    def _outer(j, _):
        jb = lax.rem(j, 2)
        sidx = src_idx.at[jb]
        didx = dst_idx.at[jb]

        def _g(h):
            return pltpu.async_copy(xc.at[sidx.at[h]], halves[h % 4],
                                    gsems4[h % 4])

        gd = {}
        sd = {}
        for h in range(4):
            gd[h] = _g(h)

        nj = j + 1

        @pl.when(nj < OUTER)
        def _():
            jn = lax.rem(nj, 2)
            pltpu.async_copy(src_hbm.at[s, nj], src_idx.at[jn], isem).wait()
            pltpu.async_copy(dst_hbm.at[s, nj], dst_idx.at[jn], isem).wait()

        for k in range(8):
            cur = bufs[k % 2]
            gd[2 * k].wait()
            gd[2 * k + 1].wait()
            sd[k] = pltpu.async_copy(cur, acc.at[didx.at[k]],
                                     ssems[k % 2], add=True)
            if k + 2 < 8:
                sd[k].wait()
                gd[2 * k + 4] = _g(2 * k + 4)
                gd[2 * k + 5] = _g(2 * k + 5)
        sd[6].wait()
        sd[7].wait()
        return 0

    lax.fori_loop(0, OUTER, _outer, 0)

    plsc.subcore_barrier()

    for z in range(ROWS_PER_TILE // CHUNK):
        pltpu.sync_copy(acc.at[pl.ds(rbase + z * CHUNK, CHUNK)], rows_a)

        @pl.when(c == 0)
        def _():
            pltpu.sync_copy(rows_a, agg0_hbm.at[s, pl.ds(z * CHUNK, CHUNK)])

        @pl.when(c == 1)
        def _():
            pltpu.sync_copy(rows_a, agg1_hbm.at[s, pl.ds(z * CHUNK, CHUNK)])


def _sc_count_kernel(dst_hbm, cnt0_hbm, cnt1_hbm,
                     dst_idx, rows, cacc, csem):
    c = lax.axis_index("c")
    s = lax.axis_index("s")

    zeros16 = jnp.zeros((16,), jnp.float32)
    ones16 = jnp.ones((16,), jnp.float32)

    def _fill_zeros(i, _):
        for k in range(HALF // 16):
            rows[i, pl.ds(k * 16, 16)] = zeros16
        return 0
    lax.fori_loop(0, CHUNK, _fill_zeros, 0)

    rbase = s * ROWS_PER_TILE
    for z in range(ROWS_PER_TILE // CHUNK):
        pltpu.sync_copy(rows, cacc.at[pl.ds(rbase + z * CHUNK, CHUNK)])

    def _fill_ones(i, _):
        for k in range(HALF // 16):
            rows[i, pl.ds(k * 16, 16)] = ones16
        return 0
    lax.fori_loop(0, CHUNK, _fill_ones, 0)

    plsc.subcore_barrier()

    def _outer(j, _):
        pltpu.sync_copy(dst_hbm.at[c, s, j], dst_idx)
        sds = [pltpu.async_copy(rows, cacc.at[dst_idx.at[k]], csem, add=True)
               for k in range(8)]
        for sd in sds:
            sd.wait()
        return 0

    lax.fori_loop(0, COUTER, _outer, 0)

    plsc.subcore_barrier()

    for z in range(ROWS_PER_TILE // CHUNK):
        pltpu.sync_copy(cacc.at[pl.ds(rbase + z * CHUNK, CHUNK)], rows)

        @pl.when(c == 0)
        def _():
            pltpu.sync_copy(rows, cnt0_hbm.at[s, pl.ds(z * CHUNK, CHUNK)])

        @pl.when(c == 1)
        def _():
            pltpu.sync_copy(rows, cnt1_hbm.at[s, pl.ds(z * CHUNK, CHUNK)])


@jax.jit
def _sc_aggregate(x, src4d, dst4d, dst5d):
    mesh = plsc.VectorSubcoreMesh(core_axis_name="c", subcore_axis_name="s")
    agg = pl.kernel(
        _sc_agg_kernel,
        mesh=mesh,
        out_type=[
            jax.ShapeDtypeStruct((NUM_SUBCORES, ROWS_PER_TILE, HALF), jnp.float32),
            jax.ShapeDtypeStruct((NUM_SUBCORES, ROWS_PER_TILE, HALF), jnp.float32),
        ],
        scratch_types=[
            pltpu.VMEM((2, 16, CHUNK // 2), jnp.int32),
            pltpu.VMEM((2, 8, CHUNK), jnp.int32),
            pltpu.VMEM((CHUNK, HALF), jnp.float32),
            pltpu.VMEM((CHUNK, HALF), jnp.float32),
            pltpu.VMEM_SHARED((N_ACC, HALF), jnp.float32),
            pltpu.SemaphoreType.DMA,
            pltpu.SemaphoreType.DMA,
            pltpu.SemaphoreType.DMA,
            pltpu.SemaphoreType.DMA,
            pltpu.SemaphoreType.DMA,
            pltpu.SemaphoreType.DMA,
            pltpu.SemaphoreType.DMA,
        ],
    )
    cnt = pl.kernel(
        _sc_count_kernel,
        mesh=mesh,
        out_type=[
            jax.ShapeDtypeStruct((NUM_SUBCORES, ROWS_PER_TILE, HALF), jnp.float32),
            jax.ShapeDtypeStruct((NUM_SUBCORES, ROWS_PER_TILE, HALF), jnp.float32),
        ],
        scratch_types=[
            pltpu.VMEM((8, CHUNK), jnp.int32),
            pltpu.VMEM((CHUNK, HALF), jnp.float32),
            pltpu.VMEM_SHARED((N_ACC, HALF), jnp.float32),
            pltpu.SemaphoreType.DMA,
        ],
    )
    agg0, agg1 = agg(x, src4d, dst4d)
    cnt0, cnt1 = cnt(dst5d)
    return agg0, agg1, cnt0, cnt1


def _tc_dense_kernel(x_ref, a0_ref, a1_ref, c0_ref, c1_ref,
                     wl_ref, bl_ref, wr_ref, out_ref):
    cnt = c0_ref[:, 0:1] + c1_ref[:, 0:1]
    denom = jnp.maximum(cnt, 1.0)
    agg = jnp.concatenate([a0_ref[...], a1_ref[...]], axis=1) / denom
    dn = (((1,), (1,)), ((), ()))
    out = lax.dot_general(agg, wl_ref[...], dn,
                          preferred_element_type=jnp.float32)
    out += lax.dot_general(x_ref[...], wr_ref[...], dn,
                           preferred_element_type=jnp.float32)
    out += bl_ref[...]
    out_ref[...] = jnp.maximum(out, 0.0)


@jax.jit
def _tc_dense(x, agg0, agg1, cnt0, cnt1, W_l, b_l2d, W_r):
    grid = 10
    bn = N_NODES // grid
    return pl.pallas_call(
        _tc_dense_kernel,
        grid=(grid,),
        in_specs=[
            pl.BlockSpec((bn, D), lambda i: (i, 0)),
            pl.BlockSpec((bn, HALF), lambda i: (i, 0)),
            pl.BlockSpec((bn, HALF), lambda i: (i, 0)),
            pl.BlockSpec((bn, HALF), lambda i: (i, 0)),
            pl.BlockSpec((bn, HALF), lambda i: (i, 0)),
            pl.BlockSpec((D, D), lambda i: (0, 0)),
            pl.BlockSpec((1, D), lambda i: (0, 0)),
            pl.BlockSpec((D, D), lambda i: (0, 0)),
        ],
        out_specs=pl.BlockSpec((bn, D), lambda i: (i, 0)),
        out_shape=jax.ShapeDtypeStruct((N_NODES, D), jnp.float32),
    )(x, agg0, agg1, cnt0, cnt1, W_l, b_l2d, W_r)


def kernel(x, edge_index, W_l, b_l, W_r):
    ei = edge_index.astype(jnp.int32)
    npad = E_PAD - E
    src = jnp.concatenate([ei[0], jnp.zeros((npad,), jnp.int32)])
    dst = jnp.concatenate([ei[1], jnp.full((npad,), N_NODES, jnp.int32)])
    src4d = src.reshape(NUM_SUBCORES, OUTER, 16, CHUNK // 2)
    dst4d = dst.reshape(NUM_SUBCORES, OUTER, 8, CHUNK)
    dst5d = dst.reshape(2, NUM_SUBCORES, COUTER, 8, CHUNK)
    agg0, agg1, cnt0, cnt1 = _sc_aggregate(x, src4d, dst4d, dst5d)
    agg0 = agg0.reshape(N_ACC, HALF)[:N_NODES]
    agg1 = agg1.reshape(N_ACC, HALF)[:N_NODES]
    cnt0 = cnt0.reshape(N_ACC, HALF)[:N_NODES]
    cnt1 = cnt1.reshape(N_ACC, HALF)[:N_NODES]
    return _tc_dense(x, agg0, agg1, cnt0, cnt1, W_l, b_l.reshape(1, D), W_r)
